# baseline (speedup 1.0000x reference)
import jax
import jax.numpy as jnp
from jax.experimental import pallas as pl
from jax.experimental.pallas import tpu as pltpu

LANE = 128


def _linear_t_kernel(x_ref, w_ref, b_ref, o_ref):
    acc = jax.lax.dot_general(
        w_ref[...], x_ref[...],
        dimension_numbers=(((1,), (1,)), ((), ())),
        preferred_element_type=jnp.float32)
    o_ref[...] = (acc + b_ref[:, 0:1]).astype(o_ref.dtype)


def kernel(x, w, b):
    B, D_in = x.shape
    D_out = w.shape[0]

    b_p = jnp.zeros((D_out, LANE), x.dtype).at[:, 0].set(b.astype(x.dtype))

    tb = 4096
    n_tiles = B // tb

    out_t = pl.pallas_call(
        _linear_t_kernel,
        out_shape=jax.ShapeDtypeStruct((D_out, B), x.dtype),
        grid_spec=pltpu.PrefetchScalarGridSpec(
            num_scalar_prefetch=0,
            grid=(n_tiles,),
            in_specs=[
                pl.BlockSpec((tb, D_in), lambda i: (i, 0)),
                pl.BlockSpec((D_out, D_in), lambda i: (0, 0)),
                pl.BlockSpec((D_out, LANE), lambda i: (0, 0)),
            ],
            out_specs=pl.BlockSpec((D_out, tb), lambda i: (0, i)),
        ),
        compiler_params=pltpu.CompilerParams(
            dimension_semantics=("parallel",),
        ),
    )(x, w, b_p)
    return out_t.T
